# baseline (speedup 1.0000x reference)
import jax
import jax.numpy as jnp
from jax.experimental import pallas as pl
from jax.experimental.pallas import tpu as pltpu

SLOTS = 65536
D = 256
B = 4096
NJ = SLOTS // B

_DN = (((1,), (0,)), ((), ()))


def _pass_a(c_ref, keys_ref, sims_ref, m_ref, m_s):
    j = pl.program_id(0)
    sims = jax.lax.dot_general(keys_ref[...], c_ref[...], _DN,
                               preferred_element_type=jnp.float32)
    sims_ref[...] = sims
    bmax = jnp.max(sims)
    prev = jnp.where(j == 0, -jnp.inf, m_s[0])
    m_s[0] = jnp.maximum(prev, bmax)

    @pl.when(j == NJ - 1)
    def _emit():
        m_ref[...] = m_s[0].reshape(1, 1)


def _pass_b(sims_ref, m_ref, vals_ref, alpha_ref, vhat_ref, alpha_s, acc_v):
    j = pl.program_id(0)

    @pl.when(j == 0)
    def _softmax():
        e = jnp.exp(sims_ref[...] - m_ref[0, 0])
        a = e * (1.0 / jnp.sum(e))
        alpha_s[...] = a
        alpha_ref[...] = a
        acc_v[...] = jnp.zeros((1, D), jnp.float32)

    ab = alpha_s[pl.ds(j, 1)].reshape(1, B)
    part = jax.lax.dot_general(ab, vals_ref[...], _DN,
                               preferred_element_type=jnp.float32)
    acc_v[...] += part

    @pl.when(j == NJ - 1)
    def _emit():
        vhat_ref[...] = acc_v[...]


@jax.jit
def kernel(c, keys, vals):
    c2 = c.reshape(D, 1)

    sims, m = pl.pallas_call(
        _pass_a,
        grid=(NJ,),
        in_specs=[
            pl.BlockSpec((D, 1), lambda j: (0, 0)),
            pl.BlockSpec((B, D), lambda j: (j, 0)),
        ],
        out_specs=[
            pl.BlockSpec((B, 1), lambda j: (j, 0)),
            pl.BlockSpec((1, 1), lambda j: (0, 0)),
        ],
        out_shape=[
            jax.ShapeDtypeStruct((SLOTS, 1), jnp.float32),
            jax.ShapeDtypeStruct((1, 1), jnp.float32),
        ],
        scratch_shapes=[pltpu.SMEM((1,), jnp.float32)],
        compiler_params=pltpu.CompilerParams(
            dimension_semantics=("arbitrary",),
        ),
    )(c2, keys)

    sims3 = sims.reshape(NJ, 1, B)
    alpha3, vhat2 = pl.pallas_call(
        _pass_b,
        grid=(NJ,),
        in_specs=[
            pl.BlockSpec((NJ, 1, B), lambda j: (0, 0, 0)),
            pl.BlockSpec((1, 1), lambda j: (0, 0)),
            pl.BlockSpec((B, D), lambda j: (j, 0)),
        ],
        out_specs=[
            pl.BlockSpec((NJ, 1, B), lambda j: (0, 0, 0)),
            pl.BlockSpec((1, D), lambda j: (0, 0)),
        ],
        out_shape=[
            jax.ShapeDtypeStruct((NJ, 1, B), jnp.float32),
            jax.ShapeDtypeStruct((1, D), jnp.float32),
        ],
        scratch_shapes=[
            pltpu.VMEM((NJ, 1, B), jnp.float32),
            pltpu.VMEM((1, D), jnp.float32),
        ],
        compiler_params=pltpu.CompilerParams(
            dimension_semantics=("arbitrary",),
        ),
    )(sims3, m, vals)
    return (vhat2.reshape(D), alpha3.reshape(SLOTS))


# speedup vs baseline: 1.0684x; 1.0684x over previous
import jax
import jax.numpy as jnp
from jax.experimental import pallas as pl
from jax.experimental.pallas import tpu as pltpu

SLOTS = 65536
D = 256
B = 8192
NJ = SLOTS // B

_DN = (((1,), (0,)), ((), ()))


def _pass_a(c_ref, keys_ref, sims_ref, m_ref, m_s):
    j = pl.program_id(0)
    sims = jax.lax.dot_general(keys_ref[...], c_ref[...], _DN,
                               preferred_element_type=jnp.float32)
    sims_ref[...] = sims
    bmax = jnp.max(sims)
    prev = jnp.where(j == 0, -jnp.inf, m_s[0])
    m_s[0] = jnp.maximum(prev, bmax)

    @pl.when(j == NJ - 1)
    def _emit():
        m_ref[...] = m_s[0].reshape(1, 1)


def _pass_b(sims_ref, m_ref, vals_ref, alpha_ref, vhat_ref, alpha_s, acc_v):
    j = pl.program_id(0)

    @pl.when(j == 0)
    def _softmax():
        e = jnp.exp(sims_ref[...] - m_ref[0, 0])
        a = e * (1.0 / jnp.sum(e))
        alpha_s[...] = a
        alpha_ref[...] = a
        acc_v[...] = jnp.zeros((1, D), jnp.float32)

    ab = alpha_s[pl.ds(j, 1)].reshape(1, B)
    part = jax.lax.dot_general(ab, vals_ref[...], _DN,
                               preferred_element_type=jnp.float32)
    acc_v[...] += part

    @pl.when(j == NJ - 1)
    def _emit():
        vhat_ref[...] = acc_v[...]


@jax.jit
def kernel(c, keys, vals):
    c2 = c.reshape(D, 1)

    sims, m = pl.pallas_call(
        _pass_a,
        grid=(NJ,),
        in_specs=[
            pl.BlockSpec((D, 1), lambda j: (0, 0)),
            pl.BlockSpec((B, D), lambda j: (j, 0)),
        ],
        out_specs=[
            pl.BlockSpec((B, 1), lambda j: (j, 0)),
            pl.BlockSpec((1, 1), lambda j: (0, 0)),
        ],
        out_shape=[
            jax.ShapeDtypeStruct((SLOTS, 1), jnp.float32),
            jax.ShapeDtypeStruct((1, 1), jnp.float32),
        ],
        scratch_shapes=[pltpu.SMEM((1,), jnp.float32)],
        compiler_params=pltpu.CompilerParams(
            dimension_semantics=("arbitrary",),
        ),
    )(c2, keys)

    sims3 = sims.reshape(NJ, 1, B)
    alpha3, vhat2 = pl.pallas_call(
        _pass_b,
        grid=(NJ,),
        in_specs=[
            pl.BlockSpec((NJ, 1, B), lambda j: (0, 0, 0)),
            pl.BlockSpec((1, 1), lambda j: (0, 0)),
            pl.BlockSpec((B, D), lambda j: (j, 0)),
        ],
        out_specs=[
            pl.BlockSpec((NJ, 1, B), lambda j: (0, 0, 0)),
            pl.BlockSpec((1, D), lambda j: (0, 0)),
        ],
        out_shape=[
            jax.ShapeDtypeStruct((NJ, 1, B), jnp.float32),
            jax.ShapeDtypeStruct((1, D), jnp.float32),
        ],
        scratch_shapes=[
            pltpu.VMEM((NJ, 1, B), jnp.float32),
            pltpu.VMEM((1, D), jnp.float32),
        ],
        compiler_params=pltpu.CompilerParams(
            dimension_semantics=("arbitrary",),
        ),
    )(sims3, m, vals)
    return (vhat2.reshape(D), alpha3.reshape(SLOTS))
